# baseline (speedup 1.0000x reference)
import jax
import jax.numpy as jnp
from jax.experimental import pallas as pl
from jax.experimental.pallas import tpu as pltpu

_BLOCK_ROWS = 4096


def _copy_kernel(src_ref, dst_ref):
    dst_ref[...] = src_ref[...]


def kernel(positions, pos_embedding):
    seq_len = positions.shape[1]
    d_model = pos_embedding.shape[2]
    grid = (seq_len // _BLOCK_ROWS,)
    return pl.pallas_call(
        _copy_kernel,
        grid=grid,
        out_shape=jax.ShapeDtypeStruct((1, seq_len, d_model), pos_embedding.dtype),
        in_specs=[
            pl.BlockSpec((1, _BLOCK_ROWS, d_model), lambda i: (0, i, 0)),
        ],
        out_specs=pl.BlockSpec((1, _BLOCK_ROWS, d_model), lambda i: (0, i, 0)),
    )(pos_embedding)
